# Initial kernel scaffold; baseline (speedup 1.0000x reference)
import jax
import jax.numpy as jnp
from jax.experimental import pallas as pl


def kernel(x, edge_index, W1, a_src1, a_dst1, b1, W2, a_src2, a_dst2, b2, W3, a_src3, a_dst3, b3, Wl1, bl1, Wl2, bl2):
    raise NotImplementedError("write your pallas kernel here")


# speedup vs baseline: 23.4559x; 23.4559x over previous
import functools

import jax
import jax.numpy as jnp
from jax import lax
from jax.experimental import pallas as pl
from jax.experimental.pallas import tpu as pltpu
from jax.experimental.pallas import tpu_sc as plsc

N = 10000
E = 320000
D = 128
DOUT = 64
NP = 10240
TRASH = N

NC, NS, L = 2, 16, 16
NWORK = NC * NS
K = 128
G = 81
EW = G * K
EPAD = NWORK * EW
RPT = NP // NS

RB = 1024
GB = NP // RB


def _attn_tail(hb, A_ref, h_ref, as_ref, ad_ref, c_ref, mx_ref, step):
    sa = jnp.dot(hb, A_ref[...], preferred_element_type=jnp.float32)
    h_ref[...] = hb
    as_b = sa[:, 0]
    ad_b = sa[:, 1]
    as_ref[...] = as_b
    ad_ref[...] = ad_b

    @pl.when(step == 0)
    def _():
        mx_ref[0] = -jnp.inf
        mx_ref[1] = -jnp.inf

    mx_ref[0] = jnp.maximum(mx_ref[0], jnp.max(as_b))
    mx_ref[1] = jnp.maximum(mx_ref[1], jnp.max(ad_b))
    c = jnp.maximum(mx_ref[0] + mx_ref[1], 0.0)
    c_ref[...] = jnp.full((L,), c, jnp.float32)


def _finalize(np_ref, den_ref, b_ref):
    nb = np_ref[0] + np_ref[1]
    den = jnp.sum(den_ref[...], axis=0)
    return jax.nn.relu(nb / (den[:, None] + 1e-16) + b_ref[...])


def _pre_first_body(x_ref, W_ref, A_ref, h_ref, as_ref, ad_ref, c_ref, mx_ref):
    i = pl.program_id(0)
    hb = jnp.dot(x_ref[...], W_ref[...], preferred_element_type=jnp.float32)
    _attn_tail(hb, A_ref, h_ref, as_ref, ad_ref, c_ref, mx_ref, i)


def _pre_mid_body(np_ref, den_ref, b_ref, W_ref, A_ref, h_ref, as_ref, ad_ref,
                  c_ref, mx_ref):
    i = pl.program_id(0)
    xb = _finalize(np_ref, den_ref, b_ref)
    hb = jnp.dot(xb, W_ref[...], preferred_element_type=jnp.float32)
    _attn_tail(hb, A_ref, h_ref, as_ref, ad_ref, c_ref, mx_ref, i)


_PRE_OUT = [
    jax.ShapeDtypeStruct((NP, D), jnp.float32),
    jax.ShapeDtypeStruct((NP,), jnp.float32),
    jax.ShapeDtypeStruct((NP,), jnp.float32),
    jax.ShapeDtypeStruct((L,), jnp.float32),
]
_PRE_OUT_SPECS = [
    pl.BlockSpec((RB, D), lambda i: (i, 0)),
    pl.BlockSpec((RB,), lambda i: (i,)),
    pl.BlockSpec((RB,), lambda i: (i,)),
    pl.BlockSpec((L,), lambda i: (0,)),
]
_NP_SPEC = pl.BlockSpec((NC, RB, D), lambda i: (0, i, 0))
_DEN_SPEC = pl.BlockSpec((NWORK, RB), lambda i: (0, i))
_W_SPEC = pl.BlockSpec((D, D), lambda i: (0, 0))
_A_SPEC = pl.BlockSpec((D, 2), lambda i: (0, 0))
_B_SPEC = pl.BlockSpec((D,), lambda i: (0,))

_pre_first = pl.pallas_call(
    _pre_first_body,
    grid=(GB,),
    in_specs=[pl.BlockSpec((RB, D), lambda i: (i, 0)), _W_SPEC, _A_SPEC],
    out_specs=_PRE_OUT_SPECS,
    out_shape=_PRE_OUT,
    scratch_shapes=[pltpu.SMEM((2,), jnp.float32)],
)

_pre_mid = pl.pallas_call(
    _pre_mid_body,
    grid=(GB,),
    in_specs=[_NP_SPEC, _DEN_SPEC, _B_SPEC, _W_SPEC, _A_SPEC],
    out_specs=_PRE_OUT_SPECS,
    out_shape=_PRE_OUT,
    scratch_shapes=[pltpu.SMEM((2,), jnp.float32)],
)


def _head_body(np_ref, den_ref, b_ref, Wl1_ref, bl1_ref, Wl2_ref, bl2_ref,
               xo_ref, out_ref):
    xo = _finalize(np_ref, den_ref, b_ref)
    xo_ref[...] = xo
    z = jax.nn.relu(
        jnp.dot(xo, Wl1_ref[...], preferred_element_type=jnp.float32)
        + bl1_ref[...])
    out_ref[...] = jax.nn.sigmoid(
        jnp.dot(z, Wl2_ref[...], preferred_element_type=jnp.float32)
        + bl2_ref[...])


_head = pl.pallas_call(
    _head_body,
    grid=(GB,),
    in_specs=[_NP_SPEC, _DEN_SPEC, _B_SPEC,
              _W_SPEC, _B_SPEC,
              pl.BlockSpec((D, DOUT), lambda i: (0, 0)),
              pl.BlockSpec((DOUT,), lambda i: (0,))],
    out_specs=[pl.BlockSpec((RB, D), lambda i: (i, 0)),
               pl.BlockSpec((RB, DOUT), lambda i: (i, 0))],
    out_shape=[jax.ShapeDtypeStruct((NP, D), jnp.float32),
               jax.ShapeDtypeStruct((NP, DOUT), jnp.float32)],
)


def _sc_edge_body(h_hbm, src_hbm, dst_hbm, as_hbm, ad_hbm, c_hbm,
                  out_hbm, den_hbm,
                  acc, sidx, didx, rows, w_v, as_v, ad_v, den_v, cv, sem):
    cid = lax.axis_index("c")
    sid = lax.axis_index("s")
    wid = cid * NS + sid
    zero16 = jnp.zeros((L,), jnp.float32)

    @pl.loop(0, K)
    def _(r):
        for q in range(D // L):
            rows[r, pl.ds(q * L, L)] = zero16

    for t in range(RPT // K):
        pltpu.sync_copy(rows, acc.at[pl.ds(sid * RPT + t * K, K)])

    @pl.loop(0, NP // L)
    def _(r):
        den_v[pl.ds(r * L, L)] = zero16

    pltpu.sync_copy(as_hbm, as_v)
    pltpu.sync_copy(ad_hbm, ad_v)
    pltpu.sync_copy(c_hbm, cv)
    cvec = cv[...]
    plsc.subcore_barrier()

    ebase = wid * EW

    @pl.loop(0, G)
    def _(g):
        base = ebase + g * K
        pltpu.sync_copy(src_hbm.at[pl.ds(base, K)], sidx)
        pltpu.sync_copy(dst_hbm.at[pl.ds(base, K)], didx)
        pltpu.async_copy(h_hbm.at[sidx], rows, sem).wait()

        for i in range(K // L):
            si = sidx[pl.ds(i * L, L)]
            di = didx[pl.ds(i * L, L)]
            e = plsc.load_gather(as_v, [si]) + plsc.load_gather(ad_v, [di])
            e = jnp.where(e > 0, e, 0.2 * e)
            w = jnp.exp(e - cvec)
            w_v[pl.ds(i * L, L)] = w
            plsc.addupdate_scatter(den_v, [di], w)

        @pl.loop(0, K)
        def _(j):
            wb = plsc.load_gather(w_v, [jnp.zeros((L,), jnp.int32) + j])
            for q in range(D // L):
                rows[j, pl.ds(q * L, L)] = rows[j, pl.ds(q * L, L)] * wb

        pltpu.sync_copy(rows, acc.at[didx], add=True)

    plsc.subcore_barrier()
    pltpu.sync_copy(acc.at[pl.ds(sid * RPT, RPT)],
                    out_hbm.at[cid, pl.ds(sid * RPT, RPT)])
    pltpu.sync_copy(den_v, den_hbm.at[wid])


@functools.cache
def _sc_edge_kernel():
    return pl.kernel(
        _sc_edge_body,
        out_type=[jax.ShapeDtypeStruct((NC, NP, D), jnp.float32),
                  jax.ShapeDtypeStruct((NWORK, NP), jnp.float32)],
        mesh=plsc.VectorSubcoreMesh(core_axis_name="c", subcore_axis_name="s",
                                    num_cores=NC, num_subcores=NS),
        scratch_types=[
            pltpu.VMEM_SHARED((NP, D), jnp.float32),
            pltpu.VMEM((K,), jnp.int32),
            pltpu.VMEM((K,), jnp.int32),
            pltpu.VMEM((K, D), jnp.float32),
            pltpu.VMEM((K,), jnp.float32),
            pltpu.VMEM((NP,), jnp.float32),
            pltpu.VMEM((NP,), jnp.float32),
            pltpu.VMEM((NP,), jnp.float32),
            pltpu.VMEM((L,), jnp.float32),
            pltpu.SemaphoreType.DMA,
        ],
        compiler_params=pltpu.CompilerParams(needs_layout_passes=False),
    )


def kernel(x, edge_index, W1, a_src1, a_dst1, b1, W2, a_src2, a_dst2, b2,
           W3, a_src3, a_dst3, b3, Wl1, bl1, Wl2, bl2):
    _sc_edge = _sc_edge_kernel()
    loop = jnp.arange(N, dtype=edge_index.dtype)
    padi = jnp.full((EPAD - E - N,), TRASH, edge_index.dtype)
    src = jnp.concatenate([edge_index[0], loop, padi])
    dst = jnp.concatenate([edge_index[1], loop, padi])
    x_pad = jnp.pad(x, ((0, NP - N), (0, 0)))

    h, as_, ad, c = _pre_first(x_pad, W1, jnp.stack([a_src1, a_dst1], axis=1))
    npart, dpart = _sc_edge(h, src, dst, as_, ad, c)
    h, as_, ad, c = _pre_mid(npart, dpart, b1, W2,
                             jnp.stack([a_src2, a_dst2], axis=1))
    npart, dpart = _sc_edge(h, src, dst, as_, ad, c)
    h, as_, ad, c = _pre_mid(npart, dpart, b2, W3,
                             jnp.stack([a_src3, a_dst3], axis=1))
    npart, dpart = _sc_edge(h, src, dst, as_, ad, c)
    x_out, out = _head(npart, dpart, b3, Wl1, bl1, Wl2, bl2)
    return (x_out[:N], out[:N])
